# baseline (speedup 1.0000x reference)
import functools

import jax
import jax.numpy as jnp
from jax import lax
from jax.experimental import pallas as pl
from jax.experimental.pallas import tpu as pltpu
from jax.experimental.pallas import tpu_sc as plsc

N = 10000
E = 320000
D = 128

NC = 2
NS = 16
NW = NC * NS
C = 64
NBUF = 3
K = 162
EPAD = NW * K * C
NPAD = 10112
RZ = NPAD // NS


NIB = 4


def _segment_sum_sc(x, pk, zeros):
  mesh = plsc.VectorSubcoreMesh(core_axis_name="c", subcore_axis_name="s")

  @functools.partial(
      pl.kernel,
      mesh=mesh,
      out_type=jax.ShapeDtypeStruct((NC, NPAD, D), jnp.float32),
      scratch_types=[
          pltpu.VMEM((K, C), jnp.int32),
          pltpu.VMEM((NBUF, C), jnp.int32),
          pltpu.VMEM((NBUF, C), jnp.int32),
          [pltpu.VMEM((C, D), jnp.float32)] * NBUF,
          pltpu.VMEM_SHARED((NPAD, D), jnp.float32),
          [pltpu.SemaphoreType.DMA] * NBUF,
          [pltpu.SemaphoreType.DMA] * NBUF,
      ],
  )
  def seg_kernel(x_hbm, pk_hbm, zero_hbm, out_hbm,
                 pk_v, srcb_v, dstb_v, rows, acc_sh, gsems, ssems):
    cid = lax.axis_index("c")
    sid = lax.axis_index("s")
    wid = sid * NC + cid

    pltpu.sync_copy(zero_hbm.at[pl.ds(sid * RZ, RZ)],
                    acc_sh.at[pl.ds(sid * RZ, RZ)])
    pltpu.sync_copy(pk_hbm.at[wid], pk_v)
    plsc.subcore_barrier()

    def unpack(k, b):
      for i in range(C // 16):
        v = pk_v[k, pl.ds(i * 16, 16)]
        srcb_v[b, pl.ds(i * 16, 16)] = lax.bitwise_and(v, 16383)
        dstb_v[b, pl.ds(i * 16, 16)] = lax.shift_right_logical(v, 14)


    unpack(0, 0)
    pltpu.async_copy(x_hbm.at[srcb_v.at[0]], rows[0], gsems[0])
    for kb in range(NBUF - 1):
      jn = kb + 1
      unpack(jn, jn)
      pltpu.async_copy(x_hbm.at[srcb_v.at[jn]], rows[jn], gsems[jn])
      pltpu.make_async_copy(x_hbm.at[srcb_v.at[kb]], rows[kb],
                            gsems[kb]).wait()
      pass

    @pl.loop(NBUF - 1, K - 1, step=NBUF)
    def _(k):
      for b in range(NBUF):
        kb = k + b
        j = (NBUF - 1 + b) % NBUF
        jn = (j + 1) % NBUF
        unpack(kb + 1, jn)
        pltpu.async_copy(x_hbm.at[srcb_v.at[jn]], rows[jn], gsems[jn])
        pltpu.make_async_copy(x_hbm.at[srcb_v.at[j]], rows[j],
                              gsems[j]).wait()
        pass

    jl = (K - 1) % NBUF
    pltpu.make_async_copy(x_hbm.at[srcb_v.at[jl]], rows[jl],
                          gsems[jl]).wait()
    pass

    plsc.subcore_barrier()
    pltpu.sync_copy(acc_sh.at[pl.ds(sid * RZ, RZ)],
                    out_hbm.at[cid].at[pl.ds(sid * RZ, RZ)])

  return seg_kernel(x, pk, zeros)


BN = 1000


def _combine_tc(parts, x, w_rel, b_rel, w_root):
  def body(p0_ref, p1_ref, x_ref, wrel_ref, wroot_ref, b_ref, o_ref):
    agg = p0_ref[0] + p1_ref[0]
    dn = (((1,), (1,)), ((), ()))
    rel = lax.dot_general(agg, wrel_ref[...], dn,
                          preferred_element_type=jnp.float32)
    root = lax.dot_general(x_ref[...], wroot_ref[...], dn,
                           preferred_element_type=jnp.float32)
    o_ref[...] = rel + root + b_ref[...]

  return pl.pallas_call(
      body,
      grid=(N // BN,),
      in_specs=[
          pl.BlockSpec((1, BN, D), lambda i: (0, i, 0)),
          pl.BlockSpec((1, BN, D), lambda i: (1, i, 0)),
          pl.BlockSpec((BN, D), lambda i: (i, 0)),
          pl.BlockSpec((D, D), lambda i: (0, 0)),
          pl.BlockSpec((D, D), lambda i: (0, 0)),
          pl.BlockSpec((D,), lambda i: (0,)),
      ],
      out_specs=pl.BlockSpec((BN, D), lambda i: (i, 0)),
      out_shape=jax.ShapeDtypeStruct((N, D), jnp.float32),
  )(parts, parts, x, w_rel, w_root, b_rel)


def kernel(x, edge_index, W1_rel, b1_rel, W1_root, W2_rel, b2_rel, W2_root):
  src = edge_index[0]
  dst = edge_index[1]
  pad = EPAD - E
  pad_iota = jnp.arange(pad, dtype=jnp.int32)
  srcs = jnp.concatenate([src, pad_iota % N])
  dsts = jnp.concatenate([dst, N + pad_iota % (NPAD - N)])
  pk = (srcs | (dsts << 14)).reshape(NW, K, C)
  zeros = jnp.zeros((NPAD, D), jnp.float32)

  p1 = _segment_sum_sc(x, pk, zeros)
  h = _combine_tc(p1, x, W1_rel, b1_rel, W1_root)
  p2 = _segment_sum_sc(h, pk, zeros)
  return _combine_tc(p2, h, W2_rel, b2_rel, W2_root)


# speedup vs baseline: 1.0333x; 1.0333x over previous
import functools

import jax
import jax.numpy as jnp
from jax import lax
from jax.experimental import pallas as pl
from jax.experimental.pallas import tpu as pltpu
from jax.experimental.pallas import tpu_sc as plsc

N = 10000
E = 320000
D = 128

NC = 2
NS = 16
NW = NC * NS
C = 64
NBUF = 3
K = 162
EPAD = NW * K * C
NPAD = 10112
RZ = NPAD // NS


NIB = 4


def _segment_sum_sc(x, pk, zeros):
  mesh = plsc.VectorSubcoreMesh(core_axis_name="c", subcore_axis_name="s")

  @functools.partial(
      pl.kernel,
      mesh=mesh,
      out_type=jax.ShapeDtypeStruct((NC, NPAD, D), jnp.float32),
      scratch_types=[
          pltpu.VMEM((K, C), jnp.int32),
          pltpu.VMEM((NBUF, C), jnp.int32),
          pltpu.VMEM((NBUF, C), jnp.int32),
          [pltpu.VMEM((C, D), jnp.float32)] * NBUF,
          pltpu.VMEM_SHARED((NPAD, D), jnp.float32),
          [pltpu.SemaphoreType.DMA] * NBUF,
          [pltpu.SemaphoreType.DMA] * NBUF,
      ],
  )
  def seg_kernel(x_hbm, pk_hbm, zero_hbm, out_hbm,
                 pk_v, srcb_v, dstb_v, rows, acc_sh, gsems, ssems):
    cid = lax.axis_index("c")
    sid = lax.axis_index("s")
    wid = sid * NC + cid

    pltpu.sync_copy(zero_hbm.at[pl.ds(sid * RZ, RZ)],
                    acc_sh.at[pl.ds(sid * RZ, RZ)])
    pltpu.sync_copy(pk_hbm.at[wid], pk_v)
    plsc.subcore_barrier()

    def unpack(k, b):
      for i in range(C // 16):
        v = pk_v[k, pl.ds(i * 16, 16)]
        srcb_v[b, pl.ds(i * 16, 16)] = lax.bitwise_and(v, 16383)
        dstb_v[b, pl.ds(i * 16, 16)] = lax.shift_right_logical(v, 14)


    unpack(0, 0)
    pltpu.async_copy(x_hbm.at[srcb_v.at[0]], rows[0], gsems[0])
    for kb in range(NBUF - 1):
      jn = kb + 1
      unpack(jn, jn)
      pltpu.async_copy(x_hbm.at[srcb_v.at[jn]], rows[jn], gsems[jn])
      pltpu.make_async_copy(x_hbm.at[srcb_v.at[kb]], rows[kb],
                            gsems[kb]).wait()
      pltpu.async_copy(rows[kb], acc_sh.at[dstb_v.at[kb]], ssems[kb],
                       add=True)

    @pl.loop(NBUF - 1, K - 1, step=NBUF)
    def _(k):
      for b in range(NBUF):
        kb = k + b
        j = (NBUF - 1 + b) % NBUF
        jn = (j + 1) % NBUF
        pltpu.make_async_copy(rows[jn], acc_sh.at[dstb_v.at[jn]],
                              ssems[jn]).wait()
        unpack(kb + 1, jn)
        pltpu.async_copy(x_hbm.at[srcb_v.at[jn]], rows[jn], gsems[jn])
        pltpu.make_async_copy(x_hbm.at[srcb_v.at[j]], rows[j],
                              gsems[j]).wait()
        pltpu.async_copy(rows[j], acc_sh.at[dstb_v.at[j]], ssems[j],
                         add=True)

    jl = (K - 1) % NBUF
    pltpu.make_async_copy(x_hbm.at[srcb_v.at[jl]], rows[jl],
                          gsems[jl]).wait()
    pltpu.async_copy(rows[jl], acc_sh.at[dstb_v.at[jl]], ssems[jl],
                     add=True)
    for j in range(NBUF):
      pltpu.make_async_copy(rows[j], acc_sh.at[dstb_v.at[j]],
                            ssems[j]).wait()

    plsc.subcore_barrier()
    pltpu.sync_copy(acc_sh.at[pl.ds(sid * RZ, RZ)],
                    out_hbm.at[cid].at[pl.ds(sid * RZ, RZ)])

  return seg_kernel(x, pk, zeros)


BN = 1000


def _combine_tc(parts, x, w_rel, b_rel, w_root):
  def body(p0_ref, p1_ref, x_ref, wrel_ref, wroot_ref, b_ref, o_ref):
    agg = p0_ref[0] + p1_ref[0]
    dn = (((1,), (1,)), ((), ()))
    rel = lax.dot_general(agg, wrel_ref[...], dn,
                          preferred_element_type=jnp.float32)
    root = lax.dot_general(x_ref[...], wroot_ref[...], dn,
                           preferred_element_type=jnp.float32)
    o_ref[...] = rel + root + b_ref[...]

  return pl.pallas_call(
      body,
      grid=(N // BN,),
      in_specs=[
          pl.BlockSpec((1, BN, D), lambda i: (0, i, 0)),
          pl.BlockSpec((1, BN, D), lambda i: (1, i, 0)),
          pl.BlockSpec((BN, D), lambda i: (i, 0)),
          pl.BlockSpec((D, D), lambda i: (0, 0)),
          pl.BlockSpec((D, D), lambda i: (0, 0)),
          pl.BlockSpec((D,), lambda i: (0,)),
      ],
      out_specs=pl.BlockSpec((BN, D), lambda i: (i, 0)),
      out_shape=jax.ShapeDtypeStruct((N, D), jnp.float32),
  )(parts, parts, x, w_rel, w_root, b_rel)


def kernel(x, edge_index, W1_rel, b1_rel, W1_root, W2_rel, b2_rel, W2_root):
  src = edge_index[0]
  dst = edge_index[1]
  pad = EPAD - E
  pad_iota = jnp.arange(pad, dtype=jnp.int32)
  srcs = jnp.concatenate([src, pad_iota % N])
  dsts = jnp.concatenate([dst, N + pad_iota % (NPAD - N)])
  pk = (srcs | (dsts << 14)).reshape(NW, K, C)
  zeros = jnp.zeros((NPAD, D), jnp.float32)

  p1 = _segment_sum_sc(x, pk, zeros)
  h = _combine_tc(p1, x, W1_rel, b1_rel, W1_root)
  p2 = _segment_sum_sc(h, pk, zeros)
  return _combine_tc(p2, h, W2_rel, b2_rel, W2_root)


# speedup vs baseline: 1.0727x; 1.0382x over previous
import functools

import jax
import jax.numpy as jnp
from jax import lax
from jax.experimental import pallas as pl
from jax.experimental.pallas import tpu as pltpu
from jax.experimental.pallas import tpu_sc as plsc

N = 10000
E = 320000
D = 128

NC = 2
NS = 16
NW = NC * NS
C = 64
NBUF = 3
K = 159
EPAD = NW * K * C
NPAD = 10112
RZ = NPAD // NS


NIB = 4


def _segment_sum_sc(x, pk, zeros):
  mesh = plsc.VectorSubcoreMesh(core_axis_name="c", subcore_axis_name="s")

  @functools.partial(
      pl.kernel,
      mesh=mesh,
      out_type=jax.ShapeDtypeStruct((NC, NPAD, D), jnp.float32),
      scratch_types=[
          pltpu.VMEM((K, C), jnp.int32),
          pltpu.VMEM((NBUF, C), jnp.int32),
          pltpu.VMEM((NBUF, C), jnp.int32),
          [pltpu.VMEM((C, D), jnp.float32)] * NBUF,
          pltpu.VMEM_SHARED((NPAD, D), jnp.float32),
          [pltpu.SemaphoreType.DMA] * NBUF,
          [pltpu.SemaphoreType.DMA] * NBUF,
      ],
  )
  def seg_kernel(x_hbm, pk_hbm, zero_hbm, out_hbm,
                 pk_v, srcb_v, dstb_v, rows, acc_sh, gsems, ssems):
    cid = lax.axis_index("c")
    sid = lax.axis_index("s")
    wid = sid * NC + cid

    pltpu.sync_copy(zero_hbm.at[pl.ds(sid * RZ, RZ)],
                    acc_sh.at[pl.ds(sid * RZ, RZ)])
    pltpu.sync_copy(pk_hbm.at[wid], pk_v)

    def unpack(k, b):
      for i in range(C // 16):
        v = pk_v[k, pl.ds(i * 16, 16)]
        srcb_v[b, pl.ds(i * 16, 16)] = lax.bitwise_and(v, 16383)
        dstb_v[b, pl.ds(i * 16, 16)] = lax.shift_right_logical(v, 14)


    unpack(0, 0)
    pltpu.async_copy(x_hbm.at[srcb_v.at[0]], rows[0], gsems[0])
    for kb in range(NBUF - 1):
      jn = kb + 1
      unpack(jn, jn)
      pltpu.async_copy(x_hbm.at[srcb_v.at[jn]], rows[jn], gsems[jn])
      pltpu.make_async_copy(x_hbm.at[srcb_v.at[kb]], rows[kb],
                            gsems[kb]).wait()
      if kb == 0:
        plsc.subcore_barrier()
      pltpu.async_copy(rows[kb], acc_sh.at[dstb_v.at[kb]], ssems[kb],
                       add=True)

    @pl.loop(NBUF - 1, K - 1, step=NBUF)
    def _(k):
      for b in range(NBUF):
        kb = k + b
        j = (NBUF - 1 + b) % NBUF
        jn = (j + 1) % NBUF
        pltpu.make_async_copy(rows[jn], acc_sh.at[dstb_v.at[jn]],
                              ssems[jn]).wait()
        unpack(kb + 1, jn)
        pltpu.async_copy(x_hbm.at[srcb_v.at[jn]], rows[jn], gsems[jn])
        pltpu.make_async_copy(x_hbm.at[srcb_v.at[j]], rows[j],
                              gsems[j]).wait()
        pltpu.async_copy(rows[j], acc_sh.at[dstb_v.at[j]], ssems[j],
                         add=True)

    jl = (K - 1) % NBUF
    pltpu.make_async_copy(x_hbm.at[srcb_v.at[jl]], rows[jl],
                          gsems[jl]).wait()
    pltpu.async_copy(rows[jl], acc_sh.at[dstb_v.at[jl]], ssems[jl],
                     add=True)
    for j in range(NBUF):
      pltpu.make_async_copy(rows[j], acc_sh.at[dstb_v.at[j]],
                            ssems[j]).wait()

    plsc.subcore_barrier()
    pltpu.sync_copy(acc_sh.at[pl.ds(sid * RZ, RZ)],
                    out_hbm.at[cid].at[pl.ds(sid * RZ, RZ)])

  return seg_kernel(x, pk, zeros)


BN = 2000


def _combine_tc(parts, x, w_rel, b_rel, w_root):
  def body(p0_ref, p1_ref, x_ref, wrel_ref, wroot_ref, b_ref, o_ref):
    agg = p0_ref[0] + p1_ref[0]
    dn = (((1,), (1,)), ((), ()))
    rel = lax.dot_general(agg, wrel_ref[...], dn,
                          preferred_element_type=jnp.float32)
    root = lax.dot_general(x_ref[...], wroot_ref[...], dn,
                           preferred_element_type=jnp.float32)
    o_ref[...] = rel + root + b_ref[...]

  return pl.pallas_call(
      body,
      grid=(N // BN,),
      in_specs=[
          pl.BlockSpec((1, BN, D), lambda i: (0, i, 0)),
          pl.BlockSpec((1, BN, D), lambda i: (1, i, 0)),
          pl.BlockSpec((BN, D), lambda i: (i, 0)),
          pl.BlockSpec((D, D), lambda i: (0, 0)),
          pl.BlockSpec((D, D), lambda i: (0, 0)),
          pl.BlockSpec((D,), lambda i: (0,)),
      ],
      out_specs=pl.BlockSpec((BN, D), lambda i: (i, 0)),
      out_shape=jax.ShapeDtypeStruct((N, D), jnp.float32),
  )(parts, parts, x, w_rel, w_root, b_rel)


def kernel(x, edge_index, W1_rel, b1_rel, W1_root, W2_rel, b2_rel, W2_root):
  src = edge_index[0]
  dst = edge_index[1]
  pad = EPAD - E
  pad_iota = jnp.arange(pad, dtype=jnp.int32)
  srcs = jnp.concatenate([src, pad_iota % N])
  dsts = jnp.concatenate([dst, N + pad_iota % (NPAD - N)])
  pk = (srcs | (dsts << 14)).reshape(NW, K, C)
  zeros = jnp.zeros((NPAD, D), jnp.float32)

  p1 = _segment_sum_sc(x, pk, zeros)
  h = _combine_tc(p1, x, W1_rel, b1_rel, W1_root)
  p2 = _segment_sum_sc(h, pk, zeros)
  return _combine_tc(p2, h, W2_rel, b2_rel, W2_root)


# speedup vs baseline: 1.0789x; 1.0057x over previous
import functools

import jax
import jax.numpy as jnp
from jax import lax
from jax.experimental import pallas as pl
from jax.experimental.pallas import tpu as pltpu
from jax.experimental.pallas import tpu_sc as plsc

N = 10000
E = 320000
D = 128

NC = 2
NS = 16
NW = NC * NS
C = 128
NBUF = 2
K = 80
EPAD = NW * K * C
NPAD = 10112
RZ = NPAD // NS


NIB = 4


def _segment_sum_sc(x, pk, zeros):
  mesh = plsc.VectorSubcoreMesh(core_axis_name="c", subcore_axis_name="s")

  @functools.partial(
      pl.kernel,
      mesh=mesh,
      out_type=jax.ShapeDtypeStruct((NC, NPAD, D), jnp.float32),
      scratch_types=[
          pltpu.VMEM((K, C), jnp.int32),
          pltpu.VMEM((NBUF, C), jnp.int32),
          pltpu.VMEM((NBUF, C), jnp.int32),
          [pltpu.VMEM((C, D), jnp.float32)] * NBUF,
          pltpu.VMEM_SHARED((NPAD, D), jnp.float32),
          [pltpu.SemaphoreType.DMA] * NBUF,
          [pltpu.SemaphoreType.DMA] * NBUF,
      ],
  )
  def seg_kernel(x_hbm, pk_hbm, zero_hbm, out_hbm,
                 pk_v, srcb_v, dstb_v, rows, acc_sh, gsems, ssems):
    cid = lax.axis_index("c")
    sid = lax.axis_index("s")
    wid = sid * NC + cid

    pltpu.sync_copy(zero_hbm.at[pl.ds(sid * RZ, RZ)],
                    acc_sh.at[pl.ds(sid * RZ, RZ)])
    pltpu.sync_copy(pk_hbm.at[wid], pk_v)

    def unpack(k, b):
      for i in range(C // 16):
        v = pk_v[k, pl.ds(i * 16, 16)]
        srcb_v[b, pl.ds(i * 16, 16)] = lax.bitwise_and(v, 16383)
        dstb_v[b, pl.ds(i * 16, 16)] = lax.shift_right_logical(v, 14)


    unpack(0, 0)
    pltpu.async_copy(x_hbm.at[srcb_v.at[0]], rows[0], gsems[0])
    unpack(1, 1)
    pltpu.async_copy(x_hbm.at[srcb_v.at[1]], rows[1], gsems[1])
    pltpu.make_async_copy(x_hbm.at[srcb_v.at[0]], rows[0], gsems[0]).wait()
    plsc.subcore_barrier()
    pltpu.async_copy(rows[0], acc_sh.at[dstb_v.at[0]], ssems[0], add=True)

    @pl.loop(1, K - 1, step=NBUF)
    def _(k):
      for b in range(NBUF):
        kb = k + b
        j = (1 + b) % NBUF
        jn = (j + 1) % NBUF
        pltpu.make_async_copy(rows[jn], acc_sh.at[dstb_v.at[jn]],
                              ssems[jn]).wait()
        unpack(kb + 1, jn)
        pltpu.async_copy(x_hbm.at[srcb_v.at[jn]], rows[jn], gsems[jn])
        pltpu.make_async_copy(x_hbm.at[srcb_v.at[j]], rows[j],
                              gsems[j]).wait()
        pltpu.async_copy(rows[j], acc_sh.at[dstb_v.at[j]], ssems[j],
                         add=True)

    jl = (K - 1) % NBUF
    pltpu.make_async_copy(x_hbm.at[srcb_v.at[jl]], rows[jl],
                          gsems[jl]).wait()
    pltpu.async_copy(rows[jl], acc_sh.at[dstb_v.at[jl]], ssems[jl],
                     add=True)
    for j in range(NBUF):
      pltpu.make_async_copy(rows[j], acc_sh.at[dstb_v.at[j]],
                            ssems[j]).wait()

    plsc.subcore_barrier()
    pltpu.sync_copy(acc_sh.at[pl.ds(sid * RZ, RZ)],
                    out_hbm.at[cid].at[pl.ds(sid * RZ, RZ)])

  return seg_kernel(x, pk, zeros)


BN = 2000


def _combine_tc(parts, x, w_rel, b_rel, w_root):
  def body(p0_ref, p1_ref, x_ref, wrel_ref, wroot_ref, b_ref, o_ref):
    agg = p0_ref[0] + p1_ref[0]
    dn = (((1,), (1,)), ((), ()))
    rel = lax.dot_general(agg, wrel_ref[...], dn,
                          preferred_element_type=jnp.float32)
    root = lax.dot_general(x_ref[...], wroot_ref[...], dn,
                           preferred_element_type=jnp.float32)
    o_ref[...] = rel + root + b_ref[...]

  return pl.pallas_call(
      body,
      grid=(N // BN,),
      in_specs=[
          pl.BlockSpec((1, BN, D), lambda i: (0, i, 0)),
          pl.BlockSpec((1, BN, D), lambda i: (1, i, 0)),
          pl.BlockSpec((BN, D), lambda i: (i, 0)),
          pl.BlockSpec((D, D), lambda i: (0, 0)),
          pl.BlockSpec((D, D), lambda i: (0, 0)),
          pl.BlockSpec((D,), lambda i: (0,)),
      ],
      out_specs=pl.BlockSpec((BN, D), lambda i: (i, 0)),
      out_shape=jax.ShapeDtypeStruct((N, D), jnp.float32),
  )(parts, parts, x, w_rel, w_root, b_rel)


def kernel(x, edge_index, W1_rel, b1_rel, W1_root, W2_rel, b2_rel, W2_root):
  src = edge_index[0]
  dst = edge_index[1]
  pad = EPAD - E
  pad_iota = jnp.arange(pad, dtype=jnp.int32)
  srcs = jnp.concatenate([src, pad_iota % N])
  dsts = jnp.concatenate([dst, N + pad_iota % (NPAD - N)])
  pk = (srcs | (dsts << 14)).reshape(NW, K, C)
  zeros = jnp.zeros((NPAD, D), jnp.float32)

  p1 = _segment_sum_sc(x, pk, zeros)
  h = _combine_tc(p1, x, W1_rel, b1_rel, W1_root)
  p2 = _segment_sum_sc(h, pk, zeros)
  return _combine_tc(p2, h, W2_rel, b2_rel, W2_root)
